# baseline (speedup 1.0000x reference)
import functools

import jax
import jax.numpy as jnp
from jax import lax
from jax.experimental import pallas as pl
from jax.experimental.pallas import tpu as pltpu
from jax.experimental.pallas import tpu_sc as plsc

_SCALE = 0.125
_NUM_BUCKETS = 32

_NMIN = (
    0, 1, 2, 3, 4, 5, 6, 7, 8, 9, 10, 11, 12, 13, 14, 15,
    16, 19, 21, 24, 27, 31, 35, 40, 46, 52, 59, 67, 77, 87, 99, 113,
)

_N = 4096
_B = 512
_R_SC = 512
_NC = 2
_NS = 16
_NW = _NC * _NS
_RPW = _R_SC // _NW
_WLEN = 2 * _N


def _sc_body(table_hbm, out_hbm, tab_v, w_v, sem):
    c = lax.axis_index("c")
    s = lax.axis_index("s")
    wid = s * _NC + c
    r = (4095 - wid) % 32

    pltpu.sync_copy(table_hbm, tab_v)
    tab_lo = tab_v[pl.ds(0, 16)] * _SCALE
    tab_hi = tab_v[pl.ds(16, 16)] * _SCALE

    dnums = lax.GatherDimensionNumbers(
        offset_dims=(), collapsed_slice_dims=(0,), start_index_map=(0,))

    def take16(vec, idx):
        return lax.gather(
            vec, idx[:, None], dnums, (1,),
            mode=lax.GatherScatterMode.PROMISE_IN_BOUNDS)

    def lookup(b):
        b15 = jnp.bitwise_and(b, 15)
        return jnp.where(b < 16, take16(tab_lo, b15), take16(tab_hi, b15))

    t31 = lookup(jnp.full((16,), 31, jnp.int32))
    t0 = lookup(jnp.zeros((16,), jnp.int32))

    def fill(k, carry):
        for u in range(8):
            w_v[pl.ds(128 * k + 16 * u, 16)] = t31
            w_v[pl.ds(_N + 128 * k + 16 * u, 16)] = t0
        return carry

    lax.fori_loop(0, _N // 128, fill, 0)

    lanes = lax.iota(jnp.int32, 16)
    for k in range(247, 256):
        d = (4095 - r) - (k * 16 + lanes)
        b = jnp.zeros((16,), jnp.int32)
        for bb in range(1, _NUM_BUCKETS):
            b = jnp.where(d >= _NMIN[bb], bb, b)
        w_v[pl.ds(k * 16, 16)] = lookup(b)

    GRP = 8

    def fire(n):
        pltpu.async_copy(
            w_v.at[pl.ds(4064 - 32 * n, _N)],
            out_hbm.at[wid + 32 * n],
            sem,
        )

    def retire_one():
        pltpu.make_async_copy(
            out_hbm.at[0], w_v.at[pl.ds(_N, _N)], sem).wait()

    for u in range(GRP):
        fire(u)

    def group(g, carry):
        for u in range(GRP):
            fire(GRP + g * GRP + u)
        for u in range(GRP):
            retire_one()
        return carry

    lax.fori_loop(0, _RPW // GRP - 1, group, 0)
    for u in range(GRP):
        retire_one()


_sc_bias = functools.partial(
    pl.kernel,
    mesh=plsc.VectorSubcoreMesh(core_axis_name="c", subcore_axis_name="s"),
    out_type=jax.ShapeDtypeStruct((_R_SC, _N), jnp.float32),
    compiler_params=pltpu.CompilerParams(use_tc_tiling_on_sc=False),
    scratch_types=[
        pltpu.VMEM((_NUM_BUCKETS,), jnp.float32),
        pltpu.VMEM((_WLEN,), jnp.float32),
        pltpu.SemaphoreType.DMA,
    ],
)(_sc_body)


def _band_block(table_ref, d0, t0):
    lane = jax.lax.broadcasted_iota(jnp.int32, (8, 128), 1)
    w = jnp.full((8, 128), t0, dtype=jnp.float32)
    for b in range(1, _NUM_BUCKETS):
        w = jnp.where(lane >= _NMIN[b], table_ref[b, 0] * _SCALE, w)
    w_b = jnp.broadcast_to(w[0:1, :], (_B, 128))

    row = jax.lax.broadcasted_iota(jnp.int32, (_B, _B), 0)
    col = jax.lax.broadcasted_iota(jnp.int32, (_B, _B), 1)
    idx = jnp.clip((row - col) + d0, 0, 127)
    return jnp.take_along_axis(w_b, idx, axis=1)


def _tc_body(table_ref, out_ref, const0, const31, band0, band1, sems):
    t0 = table_ref[0, 0] * _SCALE
    t_last = table_ref[_NUM_BUCKETS - 1, 0] * _SCALE

    nb = _N // _B
    r0 = _R_SC // _B
    copies = []

    def start(src, dst):
        c = pltpu.make_async_copy(src, dst, sems.at[len(copies)])
        c.start()
        copies.append(c)

    def rows(r):
        return pl.ds((r - r0) * _B, _B)

    const0[...] = jnp.full((_B, _N - _B), t0, dtype=jnp.float32)
    const31[...] = jnp.full((_B, _N - 2 * _B), t_last, dtype=jnp.float32)
    for r in range(r0, nb):
        w0 = _N - (r + 1) * _B
        if w0 > 0:
            start(const0.at[:, pl.ds(0, w0)],
                  out_ref.at[rows(r), pl.ds((r + 1) * _B, w0)])
        w31 = (r - 1) * _B
        if w31 > 0:
            start(const31.at[:, pl.ds(0, w31)],
                  out_ref.at[rows(r), pl.ds(0, w31)])

    band0[...] = _band_block(table_ref, 0, t0)
    for r in range(r0, nb):
        start(band0, out_ref.at[rows(r), pl.ds(r * _B, _B)])
    band1[...] = _band_block(table_ref, _B, t0)
    for r in range(max(r0, 1), nb):
        start(band1, out_ref.at[rows(r), pl.ds((r - 1) * _B, _B)])

    for c in copies:
        c.wait()


def _tc_bias(table):
    return pl.pallas_call(
        _tc_body,
        in_specs=[pl.BlockSpec(memory_space=pltpu.SMEM)],
        out_specs=pl.BlockSpec(memory_space=pl.ANY),
        out_shape=jax.ShapeDtypeStruct((_N - _R_SC, _N), jnp.float32),
        scratch_shapes=[
            pltpu.VMEM((_B, _N - _B), jnp.float32),
            pltpu.VMEM((_B, _N - 2 * _B), jnp.float32),
            pltpu.VMEM((_B, _B), jnp.float32),
            pltpu.VMEM((_B, _B), jnp.float32),
            pltpu.SemaphoreType.DMA((32,)),
        ],
    )(table)


@jax.jit
def kernel(x, table):
    del x
    sc_rows = _sc_bias(table.reshape(-1))
    tc_rows = _tc_bias(table)
    return jnp.concatenate([sc_rows, tc_rows], axis=0)


# speedup vs baseline: 3.6045x; 3.6045x over previous
import jax
import jax.numpy as jnp
from jax.experimental import pallas as pl
from jax.experimental.pallas import tpu as pltpu

_SCALE = 0.125
_NUM_BUCKETS = 32

_NMIN = (
    0, 1, 2, 3, 4, 5, 6, 7, 8, 9, 10, 11, 12, 13, 14, 15,
    16, 19, 21, 24, 27, 31, 35, 40, 46, 52, 59, 67, 77, 87, 99, 113,
)

_N = 4096
_H = 128
_S = _N // _H
_W = _N + (_S - 1) * _H


def _bias_kernel(table_ref, out_ref, buf, sems):
    t0 = table_ref[0, 0] * _SCALE

    lane = jax.lax.broadcasted_iota(jnp.int32, (8, 128), 1)
    w = jnp.full((8, 128), t0, dtype=jnp.float32)
    for b in range(1, _NUM_BUCKETS):
        w = jnp.where(lane >= _NMIN[b], table_ref[b, 0] * _SCALE, w)
    w_b = jnp.broadcast_to(w[0:1, :], (_H, 128))

    row = jax.lax.broadcasted_iota(jnp.int32, (_H, _W), 0)
    col = jax.lax.broadcasted_iota(jnp.int32, (_H, _W), 1)
    idx = jnp.clip(row - col + (_S - 1) * _H, 0, 127)
    buf[...] = jnp.take_along_axis(w_b, idx, axis=1)

    copies = []
    for r in range(_S):
        c = pltpu.make_async_copy(
            buf.at[:, pl.ds((_S - 1 - r) * _H, _N)],
            out_ref.at[pl.ds(r * _H, _H)],
            sems.at[r],
        )
        c.start()
        copies.append(c)
    for c in copies:
        c.wait()


@jax.jit
def kernel(x, table):
    i, j = x.shape[-2], x.shape[-1]
    return pl.pallas_call(
        _bias_kernel,
        in_specs=[pl.BlockSpec(memory_space=pltpu.SMEM)],
        out_specs=pl.BlockSpec(memory_space=pl.ANY),
        out_shape=jax.ShapeDtypeStruct((i, j), jnp.float32),
        scratch_shapes=[
            pltpu.VMEM((_H, _W), jnp.float32),
            pltpu.SemaphoreType.DMA((_S,)),
        ],
    )(table)


# speedup vs baseline: 3.7472x; 1.0396x over previous
import jax
import jax.numpy as jnp
from jax.experimental import pallas as pl
from jax.experimental.pallas import tpu as pltpu

_SCALE = 0.125
_NUM_BUCKETS = 32

_NMIN = (
    0, 1, 2, 3, 4, 5, 6, 7, 8, 9, 10, 11, 12, 13, 14, 15,
    16, 19, 21, 24, 27, 31, 35, 40, 46, 52, 59, 67, 77, 87, 99, 113,
)

_N = 4096
_H = 128
_S = _N // _H
_W = _N + (_S - 1) * _H


def _bias_kernel(table_ref, out_ref, buf, sems):
    t0 = table_ref[0, 0] * _SCALE

    lane = jax.lax.broadcasted_iota(jnp.int32, (8, 128), 1)
    w = jnp.full((8, 128), t0, dtype=jnp.float32)
    for b in range(1, _NUM_BUCKETS):
        w = jnp.where(lane >= _NMIN[b], table_ref[b, 0] * _SCALE, w)
    w_b = jnp.broadcast_to(w[0:1, :], (_H, 128))

    t31 = table_ref[_NUM_BUCKETS - 1, 0] * _SCALE

    def gather_chunk(u0):
        row = jax.lax.broadcasted_iota(jnp.int32, (_H, _H), 0)
        col = jax.lax.broadcasted_iota(jnp.int32, (_H, _H), 1)
        idx = jnp.clip(row - (col + u0) + (_S - 1) * _H, 0, 127)
        return jnp.take_along_axis(w_b, idx, axis=1)

    copies = []

    def fire(r):
        c = pltpu.make_async_copy(
            buf.at[:, pl.ds((_S - 1 - r) * _H, _N)],
            out_ref.at[pl.ds(r * _H, _H)],
            sems.at[r],
        )
        c.start()
        copies.append(c)

    buf[:, pl.ds((_S - 1) * _H, _H)] = gather_chunk((_S - 1) * _H)
    buf[:, pl.ds(_N, _W - _N)] = jnp.full((_H, _W - _N), t0, dtype=jnp.float32)
    fire(0)
    buf[:, pl.ds((_S - 2) * _H, _H)] = gather_chunk((_S - 2) * _H)
    fire(1)
    c31 = jnp.full((_H, _H), t31, dtype=jnp.float32)
    for r in range(2, _S):
        buf[:, pl.ds((_S - 1 - r) * _H, _H)] = c31
        fire(r)

    for c in copies:
        c.wait()


@jax.jit
def kernel(x, table):
    i, j = x.shape[-2], x.shape[-1]
    return pl.pallas_call(
        _bias_kernel,
        in_specs=[pl.BlockSpec(memory_space=pltpu.SMEM)],
        out_specs=pl.BlockSpec(memory_space=pl.ANY),
        out_shape=jax.ShapeDtypeStruct((i, j), jnp.float32),
        scratch_shapes=[
            pltpu.VMEM((_H, _W), jnp.float32),
            pltpu.SemaphoreType.DMA((_S,)),
        ],
    )(table)
